# baseline (speedup 1.0000x reference)
import jax
import jax.numpy as jnp
from jax import lax
from jax.experimental import pallas as pl
from jax.experimental.pallas import tpu as pltpu
from jax.experimental.pallas import tpu_sc as plsc

B, T = 16, 100
H, Wd, C = 27, 48, 3
PIX = H * Wd
FW = PIX * C
NBINS = 512
NFRAMES = B * T
GROUP = 16
NGROUPS = NFRAMES // GROUP
LOOKUP_WINDOW = 101
OUTPUT_DIM = 128
PAD = (LOOKUP_WINDOW - 1) // 2


def _sc_hist_body(frames_hbm, hist_hbm, buf, h16):
    info = plsc.get_sparse_core_info()
    nc = info.num_cores
    nw = nc * info.num_subcores
    wid = lax.axis_index("s") * nc + lax.axis_index("c")

    lane_frame = lax.iota(jnp.int32, 16) * FW
    lane_hist = lax.iota(jnp.int32, 16) * NBINS
    ones = jnp.ones((16,), jnp.int32)
    zeros = jnp.zeros((16,), jnp.int32)

    ngroups_per_w = (NGROUPS + nw - 1) // nw

    def do_group(g):
        pltpu.sync_copy(frames_hbm.at[pl.ds(g * (GROUP * FW), GROUP * FW)], buf)

        def zero_body(i, _):
            h16[pl.ds(i * 16, 16)] = zeros
            return 0
        lax.fori_loop(0, GROUP * NBINS // 16, zero_body, 0, unroll=8)

        def pix_body(p, _):
            c0 = 3 * p
            r = plsc.load_gather(buf, [lane_frame + c0])
            g_ = plsc.load_gather(buf, [lane_frame + (c0 + 1)])
            b_ = plsc.load_gather(buf, [lane_frame + (c0 + 2)])
            bins = ((r >> 5) << 6) + ((g_ >> 5) << 3) + (b_ >> 5)
            plsc.addupdate_scatter(h16, [lane_hist + bins], ones)
            return 0
        lax.fori_loop(0, PIX, pix_body, 0, unroll=4)

        pltpu.sync_copy(h16, hist_hbm.at[pl.ds(g * (GROUP * NBINS), GROUP * NBINS)])

    for i in range(ngroups_per_w):
        g = wid + i * nw
        if (i + 1) * nw <= NGROUPS:
            do_group(g)
        else:
            @pl.when(g < NGROUPS)
            def _():
                do_group(g)


def _sc_histograms(frames_flat):
    mesh = plsc.VectorSubcoreMesh(core_axis_name="c", subcore_axis_name="s")
    return pl.kernel(
        _sc_hist_body,
        out_type=jax.ShapeDtypeStruct((NFRAMES * NBINS,), jnp.int32),
        mesh=mesh,
        compiler_params=pltpu.CompilerParams(needs_layout_passes=False),
        scratch_types=[
            pltpu.VMEM((GROUP * FW,), jnp.int32),
            pltpu.VMEM((GROUP * NBINS,), jnp.int32),
        ],
    )(frames_flat)


def _tc_dense_body(hist_ref, w_ref, bias_ref, out_ref):
    h = hist_ref[0].astype(jnp.float32)
    n2 = jnp.sum(h * h, axis=1, keepdims=True)
    norm = jnp.maximum(jnp.sqrt(n2), 1e-12)
    x = h / norm

    sims = lax.dot_general(
        x, x, (((1,), (1,)), ((), ())),
        preferred_element_type=jnp.float32,
    )

    zpad = jnp.zeros((T, PAD), jnp.float32)
    r = jnp.concatenate([zpad, sims, zpad], axis=1)
    width = T + 2 * PAD

    t_idx = lax.broadcasted_iota(jnp.int32, (T, width), 0)
    for s in (1, 2, 4, 8, 16, 32, 64):
        shifted = jnp.concatenate([r[:, s:], r[:, :s]], axis=1)
        r = jnp.where((t_idx & s) != 0, shifted, r)

    sims_g = r[:, :LOOKUP_WINDOW]

    out = lax.dot_general(
        sims_g, w_ref[...], (((1,), (1,)), ((), ())),
        preferred_element_type=jnp.float32,
    )
    out = jnp.maximum(out + bias_ref[0][None, :], 0.0)
    out_ref[0] = out


def _tc_dense(hist, W, bias2d):
    return pl.pallas_call(
        _tc_dense_body,
        grid=(B,),
        in_specs=[
            pl.BlockSpec((1, T, NBINS), lambda i: (i, 0, 0)),
            pl.BlockSpec((OUTPUT_DIM, LOOKUP_WINDOW), lambda i: (0, 0)),
            pl.BlockSpec((1, OUTPUT_DIM), lambda i: (0, 0)),
        ],
        out_specs=pl.BlockSpec((1, T, OUTPUT_DIM), lambda i: (i, 0, 0)),
        out_shape=jax.ShapeDtypeStruct((B, T, OUTPUT_DIM), jnp.float32),
    )(hist, W, bias2d)


@jax.jit
def kernel(inputs, W, b):
    frames_flat = inputs.reshape(NFRAMES * FW)
    hist = _sc_histograms(frames_flat)
    hist3 = hist.reshape(B, T, NBINS)
    return _tc_dense(hist3, W, b.reshape(1, OUTPUT_DIM))


# speedup vs baseline: 42.8860x; 42.8860x over previous
import jax
import jax.numpy as jnp
from jax import lax
from jax.experimental import pallas as pl
from jax.experimental.pallas import tpu as pltpu
from jax.experimental.pallas import tpu_sc as plsc

B, T = 16, 100
H, Wd, C = 27, 48, 3
PIX = H * Wd
NBINS = 512
NFRAMES = B * T
GROUP = 16
NGROUPS = NFRAMES // GROUP
PIXPAD = 1408
LOOKUP_WINDOW = 101
OUTPUT_DIM = 128
PAD = (LOOKUP_WINDOW - 1) // 2


def _tc_bin_body(fr_ref, out_ref):
    ti = lax.broadcasted_iota(jnp.int32, (T, T), 0)
    tj = lax.broadcasted_iota(jnp.int32, (T, T), 1)
    eye = jnp.where(ti == tj, 1.0, 0.0)

    parts = []
    for j in range(2):
        x = fr_ref[j]
        r = x[:, 0, :, :]
        g = x[:, 1, :, :]
        b = x[:, 2, :, :]
        bins = ((r >> 5) << 6) + ((g >> 5) << 3) + (b >> 5)
        bins2 = bins.reshape(PIX, T).astype(jnp.float32)
        binsT = lax.dot_general(
            eye, bins2, (((1,), (1,)), ((), ())),
            preferred_element_type=jnp.float32,
            precision=lax.Precision.HIGHEST,
        )
        parts.append((binsT + 0.5).astype(jnp.int32))
    binsT2 = jnp.concatenate(parts, axis=0)
    zpad = jnp.zeros((2 * T, PIXPAD - PIX), jnp.int32)
    out_ref[...] = jnp.concatenate([binsT2, zpad], axis=1)


def _tc_bin(frames_t):
    return pl.pallas_call(
        _tc_bin_body,
        grid=(B // 2,),
        in_specs=[
            pl.BlockSpec((2, H, C, Wd, T), lambda i: (i, 0, 0, 0, 0)),
        ],
        out_specs=pl.BlockSpec((2 * T, PIXPAD), lambda i: (i, 0)),
        out_shape=jax.ShapeDtypeStruct((NFRAMES, PIXPAD), jnp.int32),
    )(frames_t)


def _sc_hist_body(binned_hbm, hist_hbm, buf, h16):
    info = plsc.get_sparse_core_info()
    nc = info.num_cores
    nw = nc * info.num_subcores
    wid = lax.axis_index("s") * nc + lax.axis_index("c")

    lane_frame = lax.iota(jnp.int32, 16) * PIXPAD
    lane_hist = lax.iota(jnp.int32, 16) * NBINS
    ones = jnp.ones((16,), jnp.int32)
    zeros = jnp.zeros((16,), jnp.int32)

    ngroups_per_w = (NGROUPS + nw - 1) // nw

    def do_group(g):
        pltpu.sync_copy(
            binned_hbm.at[pl.ds(g * (GROUP * PIXPAD), GROUP * PIXPAD)], buf)

        def zero_body(i, _):
            h16[pl.ds(i * 16, 16)] = zeros
            return 0
        lax.fori_loop(0, GROUP * NBINS // 16, zero_body, 0, unroll=8)

        def pix_body(p, _):
            v = plsc.load_gather(buf, [lane_frame + p])
            plsc.addupdate_scatter(h16, [lane_hist + v], ones)
            return 0
        lax.fori_loop(0, PIX, pix_body, 0, unroll=8)

        pltpu.sync_copy(h16, hist_hbm.at[pl.ds(g * (GROUP * NBINS), GROUP * NBINS)])

    for i in range(ngroups_per_w):
        g = wid + i * nw
        if (i + 1) * nw <= NGROUPS:
            do_group(g)
        else:
            @pl.when(g < NGROUPS)
            def _():
                do_group(g)


def _sc_histograms(binned_flat):
    mesh = plsc.VectorSubcoreMesh(core_axis_name="c", subcore_axis_name="s")
    return pl.kernel(
        _sc_hist_body,
        out_type=jax.ShapeDtypeStruct((NFRAMES * NBINS,), jnp.int32),
        mesh=mesh,
        compiler_params=pltpu.CompilerParams(needs_layout_passes=False),
        scratch_types=[
            pltpu.VMEM((GROUP * PIXPAD,), jnp.int32),
            pltpu.VMEM((GROUP * NBINS,), jnp.int32),
        ],
    )(binned_flat)


def _tc_dense_body(hist_ref, w_ref, bias_ref, out_ref):
    h = hist_ref[0].astype(jnp.float32)
    n2 = jnp.sum(h * h, axis=1, keepdims=True)
    norm = jnp.maximum(jnp.sqrt(n2), 1e-12)
    x = h / norm

    sims = lax.dot_general(
        x, x, (((1,), (1,)), ((), ())),
        preferred_element_type=jnp.float32,
    )

    zpad = jnp.zeros((T, PAD), jnp.float32)
    r = jnp.concatenate([zpad, sims, zpad], axis=1)
    width = T + 2 * PAD

    t_idx = lax.broadcasted_iota(jnp.int32, (T, width), 0)
    for s in (1, 2, 4, 8, 16, 32, 64):
        shifted = jnp.concatenate([r[:, s:], r[:, :s]], axis=1)
        r = jnp.where((t_idx & s) != 0, shifted, r)

    sims_g = r[:, :LOOKUP_WINDOW]

    out = lax.dot_general(
        sims_g, w_ref[...], (((1,), (1,)), ((), ())),
        preferred_element_type=jnp.float32,
    )
    out = jnp.maximum(out + bias_ref[0][None, :], 0.0)
    out_ref[0] = out


def _tc_dense(hist3d, W, bias2d):
    return pl.pallas_call(
        _tc_dense_body,
        grid=(B,),
        in_specs=[
            pl.BlockSpec((1, T, NBINS), lambda i: (i, 0, 0)),
            pl.BlockSpec((OUTPUT_DIM, LOOKUP_WINDOW), lambda i: (0, 0)),
            pl.BlockSpec((1, OUTPUT_DIM), lambda i: (0, 0)),
        ],
        out_specs=pl.BlockSpec((1, T, OUTPUT_DIM), lambda i: (i, 0, 0)),
        out_shape=jax.ShapeDtypeStruct((B, T, OUTPUT_DIM), jnp.float32),
    )(hist3d, W, bias2d)


@jax.jit
def kernel(inputs, W, b):
    frames_t = jnp.transpose(inputs, (0, 2, 4, 3, 1))
    binned = _tc_bin(frames_t)
    hist = _sc_histograms(binned.reshape(NFRAMES * PIXPAD))
    hist3d = hist.reshape(B, T, NBINS)
    return _tc_dense(hist3d, W, b.reshape(1, OUTPUT_DIM))


# speedup vs baseline: 43.1798x; 1.0069x over previous
import jax
import jax.numpy as jnp
from jax import lax
from jax.experimental import pallas as pl
from jax.experimental.pallas import tpu as pltpu
from jax.experimental.pallas import tpu_sc as plsc

B, T = 16, 100
H, Wd, C = 27, 48, 3
PIX = H * Wd
NBINS = 512
NFRAMES = B * T
GROUP = 16
NGROUPS = NFRAMES // GROUP
PIXPAD = 1408
LOOKUP_WINDOW = 101
OUTPUT_DIM = 128
PAD = (LOOKUP_WINDOW - 1) // 2


def _tc_bin_body(fr_ref, out_ref):
    ti = lax.broadcasted_iota(jnp.int32, (T, T), 0)
    tj = lax.broadcasted_iota(jnp.int32, (T, T), 1)
    eye = jnp.where(ti == tj, 1.0, 0.0)

    parts = []
    for j in range(2):
        x = fr_ref[j]
        r = x[:, 0, :, :]
        g = x[:, 1, :, :]
        b = x[:, 2, :, :]
        bins = ((r >> 5) << 6) + ((g >> 5) << 3) + (b >> 5)
        bins2 = bins.reshape(PIX, T).astype(jnp.float32)
        binsT = lax.dot_general(
            eye, bins2, (((1,), (1,)), ((), ())),
            preferred_element_type=jnp.float32,
            precision=lax.Precision.HIGHEST,
        )
        parts.append((binsT + 0.5).astype(jnp.int32))
    binsT2 = jnp.concatenate(parts, axis=0)
    zpad = jnp.zeros((2 * T, PIXPAD - PIX), jnp.int32)
    out_ref[...] = jnp.concatenate([binsT2, zpad], axis=1)


def _tc_bin(frames_t):
    return pl.pallas_call(
        _tc_bin_body,
        grid=(B // 2,),
        in_specs=[
            pl.BlockSpec((2, H, C, Wd, T), lambda i: (i, 0, 0, 0, 0)),
        ],
        out_specs=pl.BlockSpec((2 * T, PIXPAD), lambda i: (i, 0)),
        out_shape=jax.ShapeDtypeStruct((NFRAMES, PIXPAD), jnp.int32),
    )(frames_t)


def _sc_hist_body(binned_hbm, hist_hbm, buf, h16):
    info = plsc.get_sparse_core_info()
    nc = info.num_cores
    nw = nc * info.num_subcores
    wid = lax.axis_index("s") * nc + lax.axis_index("c")

    lane_frame = lax.iota(jnp.int32, 16) * PIXPAD
    lane_hist = lax.iota(jnp.int32, 16) * NBINS
    ones = jnp.ones((16,), jnp.int32)
    zeros = jnp.zeros((16,), jnp.int32)

    ngroups_per_w = (NGROUPS + nw - 1) // nw

    def do_group(g):
        in_off = pl.multiple_of(g * (GROUP * PIXPAD), GROUP * PIXPAD)
        pltpu.sync_copy(binned_hbm.at[pl.ds(in_off, GROUP * PIXPAD)], buf)

        def zero_body(i, _):
            h16[pl.ds(i * 16, 16)] = zeros
            return 0
        lax.fori_loop(0, GROUP * NBINS // 16, zero_body, 0, unroll=8)

        def pix_body(p, idxv):
            v = plsc.load_gather(buf, [idxv])
            plsc.addupdate_scatter(h16, [lane_hist + v], ones)
            return idxv + 1
        lax.fori_loop(0, PIX, pix_body, lane_frame, unroll=8)

        out_off = pl.multiple_of(g * (GROUP * NBINS), GROUP * NBINS)
        pltpu.sync_copy(h16, hist_hbm.at[pl.ds(out_off, GROUP * NBINS)])

    for i in range(ngroups_per_w):
        g = wid + i * nw
        if (i + 1) * nw <= NGROUPS:
            do_group(g)
        else:
            @pl.when(g < NGROUPS)
            def _():
                do_group(g)


def _sc_histograms(binned_flat):
    mesh = plsc.VectorSubcoreMesh(core_axis_name="c", subcore_axis_name="s")
    return pl.kernel(
        _sc_hist_body,
        out_type=jax.ShapeDtypeStruct((NFRAMES * NBINS,), jnp.int32),
        mesh=mesh,
        compiler_params=pltpu.CompilerParams(needs_layout_passes=False),
        scratch_types=[
            pltpu.VMEM((GROUP * PIXPAD,), jnp.int32),
            pltpu.VMEM((GROUP * NBINS,), jnp.int32),
        ],
    )(binned_flat)


def _tc_dense_body(hist_ref, w_ref, bias_ref, out_ref):
    h = hist_ref[0].astype(jnp.float32)
    n2 = jnp.sum(h * h, axis=1, keepdims=True)
    norm = jnp.maximum(jnp.sqrt(n2), 1e-12)
    x = h / norm

    sims = lax.dot_general(
        x, x, (((1,), (1,)), ((), ())),
        preferred_element_type=jnp.float32,
    )

    zpad = jnp.zeros((T, PAD), jnp.float32)
    r = jnp.concatenate([zpad, sims, zpad], axis=1)
    width = T + 2 * PAD

    t_idx = lax.broadcasted_iota(jnp.int32, (T, width), 0)
    for s in (1, 2, 4, 8, 16, 32, 64):
        shifted = jnp.concatenate([r[:, s:], r[:, :s]], axis=1)
        r = jnp.where((t_idx & s) != 0, shifted, r)

    sims_g = r[:, :LOOKUP_WINDOW]

    out = lax.dot_general(
        sims_g, w_ref[...], (((1,), (1,)), ((), ())),
        preferred_element_type=jnp.float32,
    )
    out = jnp.maximum(out + bias_ref[0][None, :], 0.0)
    out_ref[0] = out


def _tc_dense(hist3d, W, bias2d):
    return pl.pallas_call(
        _tc_dense_body,
        grid=(B,),
        in_specs=[
            pl.BlockSpec((1, T, NBINS), lambda i: (i, 0, 0)),
            pl.BlockSpec((OUTPUT_DIM, LOOKUP_WINDOW), lambda i: (0, 0)),
            pl.BlockSpec((1, OUTPUT_DIM), lambda i: (0, 0)),
        ],
        out_specs=pl.BlockSpec((1, T, OUTPUT_DIM), lambda i: (i, 0, 0)),
        out_shape=jax.ShapeDtypeStruct((B, T, OUTPUT_DIM), jnp.float32),
    )(hist3d, W, bias2d)


@jax.jit
def kernel(inputs, W, b):
    frames_t = jnp.transpose(inputs, (0, 2, 4, 3, 1))
    binned = _tc_bin(frames_t)
    hist = _sc_histograms(binned.reshape(NFRAMES * PIXPAD))
    hist3d = hist.reshape(B, T, NBINS)
    return _tc_dense(hist3d, W, b.reshape(1, OUTPUT_DIM))


# speedup vs baseline: 63.0982x; 1.4613x over previous
import jax
import jax.numpy as jnp
from jax import lax
from jax.experimental import pallas as pl
from jax.experimental.pallas import tpu as pltpu
from jax.experimental.pallas import tpu_sc as plsc

B, T = 16, 100
H, Wd, C = 27, 48, 3
PIX = H * Wd
NBINS = 512
NFRAMES = B * T
GROUP = 16
NGROUPS = NFRAMES // GROUP
PIXPAD = 1408
LOOKUP_WINDOW = 101
OUTPUT_DIM = 128
PAD = (LOOKUP_WINDOW - 1) // 2


def _tc_bin_body(fr_ref, out_ref):
    ti = lax.broadcasted_iota(jnp.int32, (T, T), 0)
    tj = lax.broadcasted_iota(jnp.int32, (T, T), 1)
    eye = jnp.where(ti == tj, 1.0, 0.0)

    parts = []
    for j in range(2):
        x = fr_ref[j]
        r = x[:, 0, :, :]
        g = x[:, 1, :, :]
        b = x[:, 2, :, :]
        bins = ((r >> 5) << 6) + ((g >> 5) << 3) + (b >> 5)
        bins2 = bins.reshape(PIX, T).astype(jnp.float32)
        binsT = lax.dot_general(
            eye, bins2, (((1,), (1,)), ((), ())),
            preferred_element_type=jnp.float32,
            precision=lax.Precision.HIGHEST,
        )
        parts.append((binsT + 0.5).astype(jnp.int32))
    binsT2 = jnp.concatenate(parts, axis=0)
    zpad = jnp.zeros((2 * T, PIXPAD - PIX), jnp.int32)
    out_ref[...] = jnp.concatenate([binsT2, zpad], axis=1)


def _tc_bin(frames_t):
    return pl.pallas_call(
        _tc_bin_body,
        grid=(B // 2,),
        in_specs=[
            pl.BlockSpec((2, H, C, Wd, T), lambda i: (i, 0, 0, 0, 0)),
        ],
        out_specs=pl.BlockSpec((2 * T, PIXPAD), lambda i: (i, 0)),
        out_shape=jax.ShapeDtypeStruct((NFRAMES, PIXPAD), jnp.int32),
    )(frames_t)


def _sc_hist_body(binned_hbm, hist_hbm, buf0, buf1, h0, h1,
                  si0, si1, so0, so1):
    info = plsc.get_sparse_core_info()
    nc = info.num_cores
    nw = nc * info.num_subcores
    wid = lax.axis_index("s") * nc + lax.axis_index("c")

    lane_frame = lax.iota(jnp.int32, 16) * PIXPAD
    lane_hist = lax.iota(jnp.int32, 16) * NBINS
    ones = jnp.ones((16,), jnp.int32)
    zeros = jnp.zeros((16,), jnp.int32)

    bufs, hs = [buf0, buf1], [h0, h1]
    sis, sos = [si0, si1], [so0, so1]
    ngroups_per_w = (NGROUPS + nw - 1) // nw

    def in_slice(g):
        off = pl.multiple_of(g * (GROUP * PIXPAD), GROUP * PIXPAD)
        return binned_hbm.at[pl.ds(off, GROUP * PIXPAD)]

    def out_slice(g):
        off = pl.multiple_of(g * (GROUP * NBINS), GROUP * NBINS)
        return hist_hbm.at[pl.ds(off, GROUP * NBINS)]

    def start_in(i):
        g = wid + i * nw
        pltpu.async_copy(in_slice(g), bufs[i % 2], sis[i % 2])

    def wait_in(i):
        g = wid + i * nw
        pltpu.make_async_copy(in_slice(g), bufs[i % 2], sis[i % 2]).wait()

    def start_out(i):
        g = wid + i * nw
        pltpu.async_copy(hs[i % 2], out_slice(g), sos[i % 2])

    def wait_out(i):
        g = wid + i * nw
        pltpu.make_async_copy(hs[i % 2], out_slice(g), sos[i % 2]).wait()

    def stage(i):
        buf, h16 = bufs[i % 2], hs[i % 2]
        wait_in(i)
        if i + 1 < ngroups_per_w:
            if (i + 2) * nw <= NGROUPS:
                start_in(i + 1)
            else:
                @pl.when(wid + (i + 1) * nw < NGROUPS)
                def _():
                    start_in(i + 1)
        if i >= 2:
            wait_out(i - 2)

        @plsc.parallel_loop(0, GROUP * NBINS // 16, unroll=8)
        def _(j):
            h16[pl.ds(j * 16, 16)] = zeros

        @plsc.parallel_loop(0, PIX, unroll=8)
        def _(p):
            v = plsc.load_gather(buf, [lane_frame + p])
            plsc.addupdate_scatter(h16, [lane_hist + v], ones)

        start_out(i)

    start_in(0)
    for i in range(ngroups_per_w):
        if (i + 1) * nw <= NGROUPS:
            stage(i)
        else:
            @pl.when(wid + i * nw < NGROUPS)
            def _():
                stage(i)

    wait_out(ngroups_per_w - 2)
    if ngroups_per_w * nw <= NGROUPS:
        wait_out(ngroups_per_w - 1)
    else:
        @pl.when(wid + (ngroups_per_w - 1) * nw < NGROUPS)
        def _():
            wait_out(ngroups_per_w - 1)


def _sc_histograms(binned_flat):
    mesh = plsc.VectorSubcoreMesh(core_axis_name="c", subcore_axis_name="s")
    return pl.kernel(
        _sc_hist_body,
        out_type=jax.ShapeDtypeStruct((NFRAMES * NBINS,), jnp.int32),
        mesh=mesh,
        compiler_params=pltpu.CompilerParams(needs_layout_passes=False),
        scratch_types=[
            pltpu.VMEM((GROUP * PIXPAD,), jnp.int32),
            pltpu.VMEM((GROUP * PIXPAD,), jnp.int32),
            pltpu.VMEM((GROUP * NBINS,), jnp.int32),
            pltpu.VMEM((GROUP * NBINS,), jnp.int32),
            pltpu.SemaphoreType.DMA,
            pltpu.SemaphoreType.DMA,
            pltpu.SemaphoreType.DMA,
            pltpu.SemaphoreType.DMA,
        ],
    )(binned_flat)


def _tc_dense_body(hist_ref, w_ref, bias_ref, out_ref):
    h = hist_ref[0].astype(jnp.float32)
    n2 = jnp.sum(h * h, axis=1, keepdims=True)
    norm = jnp.maximum(jnp.sqrt(n2), 1e-12)
    x = h / norm

    sims = lax.dot_general(
        x, x, (((1,), (1,)), ((), ())),
        preferred_element_type=jnp.float32,
    )

    zpad = jnp.zeros((T, PAD), jnp.float32)
    r = jnp.concatenate([zpad, sims, zpad], axis=1)
    width = T + 2 * PAD

    t_idx = lax.broadcasted_iota(jnp.int32, (T, width), 0)
    for s in (1, 2, 4, 8, 16, 32, 64):
        shifted = jnp.concatenate([r[:, s:], r[:, :s]], axis=1)
        r = jnp.where((t_idx & s) != 0, shifted, r)

    sims_g = r[:, :LOOKUP_WINDOW]

    out = lax.dot_general(
        sims_g, w_ref[...], (((1,), (1,)), ((), ())),
        preferred_element_type=jnp.float32,
    )
    out = jnp.maximum(out + bias_ref[0][None, :], 0.0)
    out_ref[0] = out


def _tc_dense(hist3d, W, bias2d):
    return pl.pallas_call(
        _tc_dense_body,
        grid=(B,),
        in_specs=[
            pl.BlockSpec((1, T, NBINS), lambda i: (i, 0, 0)),
            pl.BlockSpec((OUTPUT_DIM, LOOKUP_WINDOW), lambda i: (0, 0)),
            pl.BlockSpec((1, OUTPUT_DIM), lambda i: (0, 0)),
        ],
        out_specs=pl.BlockSpec((1, T, OUTPUT_DIM), lambda i: (i, 0, 0)),
        out_shape=jax.ShapeDtypeStruct((B, T, OUTPUT_DIM), jnp.float32),
    )(hist3d, W, bias2d)


@jax.jit
def kernel(inputs, W, b):
    frames_t = jnp.transpose(inputs, (0, 2, 4, 3, 1))
    binned = _tc_bin(frames_t)
    hist = _sc_histograms(binned.reshape(NFRAMES * PIXPAD))
    hist3d = hist.reshape(B, T, NBINS)
    return _tc_dense(hist3d, W, b.reshape(1, OUTPUT_DIM))


# speedup vs baseline: 66.3941x; 1.0522x over previous
import jax
import jax.numpy as jnp
from jax import lax
from jax.experimental import pallas as pl
from jax.experimental.pallas import tpu as pltpu
from jax.experimental.pallas import tpu_sc as plsc

B, T = 16, 100
H, Wd, C = 27, 48, 3
PIX = H * Wd
NBINS = 512
NFRAMES = B * T
GROUP = 16
NGROUPS = NFRAMES // GROUP
PIXPAD = 1408
LOOKUP_WINDOW = 101
OUTPUT_DIM = 128
PAD = (LOOKUP_WINDOW - 1) // 2


def _tc_bin_body(fr_ref, out_ref):
    ti = lax.broadcasted_iota(jnp.int32, (T, T), 0)
    tj = lax.broadcasted_iota(jnp.int32, (T, T), 1)
    eye = jnp.where(ti == tj, 1.0, 0.0)

    parts = []
    for j in range(2):
        x = fr_ref[j]
        r = x[:, 0, :, :]
        g = x[:, 1, :, :]
        b = x[:, 2, :, :]
        bins = ((r >> 5) << 6) + ((g >> 5) << 3) + (b >> 5)
        bins2 = bins.reshape(PIX, T).astype(jnp.float32)
        binsT = lax.dot_general(
            eye, bins2, (((1,), (1,)), ((), ())),
            preferred_element_type=jnp.float32,
            precision=lax.Precision.HIGHEST,
        )
        parts.append((binsT + 0.5).astype(jnp.int32))
    binsT2 = jnp.concatenate(parts, axis=0)
    zpad = jnp.zeros((2 * T, PIXPAD - PIX), jnp.int32)
    out_ref[...] = jnp.concatenate([binsT2, zpad], axis=1)


def _tc_bin(frames_t):
    return pl.pallas_call(
        _tc_bin_body,
        grid=(B // 2,),
        in_specs=[
            pl.BlockSpec((2, H, C, Wd, T), lambda i: (i, 0, 0, 0, 0)),
        ],
        out_specs=pl.BlockSpec((2 * T, PIXPAD), lambda i: (i, 0)),
        out_shape=jax.ShapeDtypeStruct((NFRAMES, PIXPAD), jnp.int32),
    )(frames_t)


def _sc_hist_body(binned_hbm, hist_hbm, buf0, buf1, h0, h1,
                  si0, si1, so0, so1):
    info = plsc.get_sparse_core_info()
    nc = info.num_cores
    nw = nc * info.num_subcores
    wid = lax.axis_index("s") * nc + lax.axis_index("c")

    lane_frame = lax.iota(jnp.int32, 16) * PIXPAD
    lane_hist = lax.iota(jnp.int32, 16) * NBINS
    ones = jnp.ones((16,), jnp.int32)
    zeros = jnp.zeros((16,), jnp.int32)

    bufs, hs = [buf0, buf1], [h0, h1]
    sis, sos = [si0, si1], [so0, so1]
    ngroups_per_w = (NGROUPS + nw - 1) // nw

    def in_slice(g):
        off = pl.multiple_of(g * (GROUP * PIXPAD), GROUP * PIXPAD)
        return binned_hbm.at[pl.ds(off, GROUP * PIXPAD)]

    def out_slice(g):
        off = pl.multiple_of(g * (GROUP * NBINS), GROUP * NBINS)
        return hist_hbm.at[pl.ds(off, GROUP * NBINS)]

    def start_in(i):
        g = wid + i * nw
        pltpu.async_copy(in_slice(g), bufs[i % 2], sis[i % 2])

    def wait_in(i):
        g = wid + i * nw
        pltpu.make_async_copy(in_slice(g), bufs[i % 2], sis[i % 2]).wait()

    def start_out(i):
        g = wid + i * nw
        pltpu.async_copy(hs[i % 2], out_slice(g), sos[i % 2])

    def wait_out(i):
        g = wid + i * nw
        pltpu.make_async_copy(hs[i % 2], out_slice(g), sos[i % 2]).wait()

    def stage(i):
        buf, h16 = bufs[i % 2], hs[i % 2]
        wait_in(i)
        if i + 1 < ngroups_per_w:
            if (i + 2) * nw <= NGROUPS:
                start_in(i + 1)
            else:
                @pl.when(wid + (i + 1) * nw < NGROUPS)
                def _():
                    start_in(i + 1)
        if i >= 2:
            wait_out(i - 2)

        @plsc.parallel_loop(0, GROUP * NBINS // 16, unroll=8)
        def _(j):
            h16[pl.ds(j * 16, 16)] = zeros

        @plsc.parallel_loop(0, PIX, unroll=8)
        def _(p):
            v = plsc.load_gather(buf, [lane_frame + p])
            plsc.addupdate_scatter(h16, [lane_hist + v], ones)

        start_out(i)

    start_in(0)
    for i in range(ngroups_per_w):
        if (i + 1) * nw <= NGROUPS:
            stage(i)
        else:
            @pl.when(wid + i * nw < NGROUPS)
            def _():
                stage(i)

    wait_out(ngroups_per_w - 2)
    if ngroups_per_w * nw <= NGROUPS:
        wait_out(ngroups_per_w - 1)
    else:
        @pl.when(wid + (ngroups_per_w - 1) * nw < NGROUPS)
        def _():
            wait_out(ngroups_per_w - 1)


def _sc_histograms(binned_flat):
    mesh = plsc.VectorSubcoreMesh(core_axis_name="c", subcore_axis_name="s")
    return pl.kernel(
        _sc_hist_body,
        out_type=jax.ShapeDtypeStruct((NFRAMES * NBINS,), jnp.int32),
        mesh=mesh,
        compiler_params=pltpu.CompilerParams(needs_layout_passes=False),
        scratch_types=[
            pltpu.VMEM((GROUP * PIXPAD,), jnp.int32),
            pltpu.VMEM((GROUP * PIXPAD,), jnp.int32),
            pltpu.VMEM((GROUP * NBINS,), jnp.int32),
            pltpu.VMEM((GROUP * NBINS,), jnp.int32),
            pltpu.SemaphoreType.DMA,
            pltpu.SemaphoreType.DMA,
            pltpu.SemaphoreType.DMA,
            pltpu.SemaphoreType.DMA,
        ],
    )(binned_flat)


def _tc_dense_body(hist_ref, w_ref, bias_ref, out_ref):
    width = T + 2 * PAD
    t_idx = lax.broadcasted_iota(jnp.int32, (T, width), 0)
    zpad = jnp.zeros((T, PAD), jnp.float32)
    wmat = w_ref[...]
    bias = bias_ref[0][None, :]

    for j in range(B):
        h = hist_ref[pl.ds(j * T, T), :].astype(jnp.float32)
        n2 = jnp.sum(h * h, axis=1, keepdims=True)
        x = h / jnp.maximum(jnp.sqrt(n2), 1e-12)
        sims = lax.dot_general(
            x, x, (((1,), (1,)), ((), ())),
            preferred_element_type=jnp.float32,
        )
        r = jnp.concatenate([zpad, sims, zpad], axis=1)
        for s in (1, 2, 4, 8, 16, 32, 64):
            shifted = jnp.concatenate([r[:, s:], r[:, :s]], axis=1)
            r = jnp.where((t_idx & s) != 0, shifted, r)
        sims_g = r[:, :LOOKUP_WINDOW]
        out = lax.dot_general(
            sims_g, wmat, (((1,), (0,)), ((), ())),
            preferred_element_type=jnp.float32,
        )
        out_ref[:, j, :] = jnp.maximum(out + bias, 0.0)


def _tc_dense(hist2d, W, bias2d):
    return pl.pallas_call(
        _tc_dense_body,
        grid=(1,),
        in_specs=[
            pl.BlockSpec((NFRAMES, NBINS), lambda i: (0, 0)),
            pl.BlockSpec((LOOKUP_WINDOW, OUTPUT_DIM), lambda i: (0, 0)),
            pl.BlockSpec((1, OUTPUT_DIM), lambda i: (0, 0)),
        ],
        out_specs=pl.BlockSpec((T, B, OUTPUT_DIM), lambda i: (0, 0, 0)),
        out_shape=jax.ShapeDtypeStruct((T, B, OUTPUT_DIM), jnp.float32),
    )(hist2d, W, bias2d)


@jax.jit
def kernel(inputs, W, b):
    frames_t = jnp.transpose(inputs, (0, 2, 4, 3, 1))
    binned = _tc_bin(frames_t)
    hist = _sc_histograms(binned.reshape(NFRAMES * PIXPAD))
    hist2d = hist.reshape(NFRAMES, NBINS)
    out_t = _tc_dense(hist2d, jnp.transpose(W), b.reshape(1, OUTPUT_DIM))
    return jnp.transpose(out_t, (1, 0, 2))
